# Initial kernel scaffold; baseline (speedup 1.0000x reference)
import jax
import jax.numpy as jnp
from jax.experimental import pallas as pl


def kernel(log_potentials, target, lengths, transition, start_transition, end_transition):
    raise NotImplementedError("write your pallas kernel here")


# speedup vs baseline: 5.8347x; 5.8347x over previous
import functools

import jax
import jax.numpy as jnp
from jax import lax
from jax.experimental import pallas as pl

_B, _T, _N = 16, 2048, 32


def _crf_body(lp_ref, tgt_ref, len_ref, tr_ref, st_ref, en_ref, out_ref):
    f32 = jnp.float32
    E = jnp.exp(tr_ref[...])
    lens = len_ref[...]
    iota = lax.broadcasted_iota(jnp.int32, (_B, _N), 1)

    lp0 = lp_ref[0]
    tg0 = tgt_ref[0]
    oh0 = (iota == tg0).astype(f32)
    alpha0 = st_ref[...] + lp0
    start_sel = jnp.sum(st_ref[...] * oh0, axis=1, keepdims=True)

    def step(t, carry):
        alpha, prevoh, em_acc, tr_acc, last_acc = carry
        lp_t = lp_ref[t]
        tg = tgt_ref[t]
        oh = (iota == tg).astype(f32)
        valid = t < lens
        vf = valid.astype(f32)

        m = jnp.max(alpha, axis=1, keepdims=True)
        e = jnp.exp(alpha - m)
        s = lax.dot(e, E, preferred_element_type=f32)
        anew = jnp.log(s) + m + lp_t
        alpha = jnp.where(valid, anew, alpha)

        em_acc = em_acc + lp_t * oh * vf
        row = lax.dot(prevoh, tr_ref[...], preferred_element_type=f32)
        tr_acc = tr_acc + row * oh * vf
        lastf = (t == lens - 1).astype(f32)
        last_acc = last_acc + oh * lastf
        return alpha, oh, em_acc, tr_acc, last_acc

    zeros = jnp.zeros((_B, _N), f32)
    alpha, _, em_acc, tr_acc, last_acc = lax.fori_loop(
        1, _T, step, (alpha0, oh0, lp0 * oh0, zeros, zeros))

    z = alpha + en_ref[...]
    m2 = jnp.max(z, axis=1, keepdims=True)
    logZ = jnp.log(jnp.sum(jnp.exp(z - m2), axis=1, keepdims=True)) + m2
    em_sum = jnp.sum(em_acc, axis=1, keepdims=True)
    tr_sum = jnp.sum(tr_acc, axis=1, keepdims=True)
    end_sel = jnp.sum(last_acc * en_ref[...], axis=1, keepdims=True)
    out_ref[...] = start_sel + em_sum + tr_sum + end_sel - logZ


@functools.partial(jax.jit, static_argnames=())
def kernel(log_potentials, target, lengths, transition, start_transition,
           end_transition):
    lp_t = jnp.swapaxes(log_potentials, 0, 1)
    tgt_t = jnp.swapaxes(target, 0, 1)[:, :, None]
    lens = lengths[:, None].astype(jnp.int32)
    st = start_transition[None, :]
    en = end_transition[None, :]
    out = pl.pallas_call(
        _crf_body,
        out_shape=jax.ShapeDtypeStruct((_B, 1), jnp.float32),
    )(lp_t, tgt_t, lens, transition, st, en)
    return out[:, 0]


# speedup vs baseline: 11.1869x; 1.9173x over previous
import functools

import jax
import jax.numpy as jnp
from jax import lax
from jax.experimental import pallas as pl
from jax.experimental.pallas import tpu as pltpu

_B, _T, _N = 16, 2048, 32
_K, _S = 128, 16
_P = _K * _B
_NBLK = 4
_PB = _P // _NBLK


def _crf_body(lp3_ref, lps_ref, lp0_ref, tprev_ref, tcur_ref, tgt0_ref,
              ubase_ref, lenp_ref, tr_ref, st_ref, en_ref, out_ref,
              m_ref, sh_ref):
    f32 = jnp.float32
    bf16 = jnp.bfloat16
    E = jnp.exp(tr_ref[...])
    Ebf = E.astype(bf16)

    lp0 = lp0_ref[...]
    iota0 = lax.broadcasted_iota(jnp.int32, (_B, _N), 1)
    oh0 = (tgt0_ref[...] == iota0).astype(f32)
    start_sel = jnp.sum(st_ref[...] * oh0, axis=1, keepdims=True)
    em0 = jnp.sum(lp0 * oh0, axis=1, keepdims=True)

    iota3bf = lax.broadcasted_iota(jnp.int32, (_PB, _S, _N), 2).astype(bf16)
    sg2 = lax.broadcasted_iota(jnp.int32, (_PB, _S), 1)
    eyeB = (lax.broadcasted_iota(jnp.int32, (_PB, _N, _N), 1)
            == lax.broadcasted_iota(jnp.int32, (_PB, _N, _N), 2)).astype(bf16)

    def block(blk, acc):
        em_sum, tr_sum, end_sel = acc
        r0 = blk * _PB
        lp3 = lp3_ref[pl.ds(r0, _PB)].astype(f32)
        ubase = ubase_ref[pl.ds(r0, _PB), :]
        lenp = lenp_ref[pl.ds(r0, _PB), :]
        val2 = (ubase + sg2 + 1) < lenp
        lastm2 = (ubase + sg2) == (lenp - 1)

        mxps = jnp.max(lp3, axis=2)
        sh_ref[pl.ds(r0, _PB), :] = jnp.sum(
            jnp.where(val2, mxps, 0.0), axis=1, keepdims=True)

        ohcur = (tcur_ref[pl.ds(r0, _PB)][:, :, None] == iota3bf).astype(bf16)
        ohprev = (tprev_ref[pl.ds(r0, _PB)][:, :, None] == iota3bf).astype(bf16)
        emps = jnp.sum((lp3 * ohcur).astype(f32), axis=2)
        rows = lax.dot(ohprev.reshape(_PB * _S, _N), tr_ref[...].astype(bf16),
                       preferred_element_type=f32).reshape(_PB, _S, _N)
        trps = jnp.sum(rows * ohcur.astype(f32), axis=2)
        endps = jnp.sum(ohprev.astype(f32) * en_ref[...][None], axis=2)

        def _per_b(x2):
            xb = x2.reshape(_PB // _B, _B, _S)
            return jnp.sum(jnp.sum(xb, axis=0), axis=1, keepdims=True)

        em_sum = em_sum + _per_b(jnp.where(val2, emps, 0.0))
        tr_sum = tr_sum + _per_b(jnp.where(val2, trps, 0.0))
        end_sel = end_sel + _per_b(jnp.where(lastm2, endps, 0.0))

        lps0 = lps_ref[0, pl.ds(r0, _PB), :].astype(f32)
        d0 = jnp.exp(lps0 - jnp.max(lps0, axis=1, keepdims=True))
        v0 = ((ubase + 1) < lenp)[:, :, None]
        m_ref[pl.ds(r0, _PB)] = jnp.where(
            v0, E[None, :, :] * d0[:, None, :], eyeB.astype(f32)).astype(bf16)

        def step(s, _):
            lps = lps_ref[s, pl.ds(r0, _PB), :].astype(f32)
            ds = jnp.exp(lps - jnp.max(lps, axis=1, keepdims=True))
            M3 = m_ref[pl.ds(r0, _PB)]
            R = lax.dot(M3.reshape(_PB * _N, _N), Ebf,
                        preferred_element_type=f32).reshape(_PB, _N, _N)
            R = R * ds[:, None, :]
            vs = ((ubase + (s + 1)) < lenp)[:, :, None]
            m_ref[pl.ds(r0, _PB)] = jnp.where(
                vs, R, M3.astype(f32)).astype(bf16)
            return 0

        lax.fori_loop(1, _S, step, 0)
        return em_sum, tr_sum, end_sel

    zero = jnp.zeros((_B, 1), f32)
    em_sum, tr_sum, end_sel = lax.fori_loop(
        0, _NBLK, block, (zero, zero, zero))

    alpha0 = st_ref[...] + lp0

    def fold(c, alpha):
        Mc = m_ref[pl.ds(c * _B, _B)].astype(f32)
        shc = sh_ref[pl.ds(c * _B, _B), :]
        m = jnp.max(alpha, axis=1, keepdims=True)
        e = jnp.exp(alpha - m)
        prod = jnp.sum(e[:, :, None] * Mc, axis=1)
        return jnp.log(prod) + m + shc

    alpha = lax.fori_loop(0, _K, fold, alpha0)

    z = alpha + en_ref[...]
    m2 = jnp.max(z, axis=1, keepdims=True)
    logZ = jnp.log(jnp.sum(jnp.exp(z - m2), axis=1, keepdims=True)) + m2

    out_ref[...] = start_sel + em0 + em_sum + tr_sum + end_sel - logZ


@functools.partial(jax.jit, static_argnames=())
def kernel(log_potentials, target, lengths, transition, start_transition,
           end_transition):
    lp_steps = jnp.concatenate(
        [log_potentials[:, 1:, :], log_potentials[:, :1, :]], axis=1)
    lp4 = jnp.swapaxes(lp_steps.reshape(_B, _K, _S, _N), 0, 1)
    lp3 = lp4.reshape(_P, _S, _N).astype(jnp.bfloat16)
    lps = jnp.transpose(lp4, (2, 0, 1, 3)).reshape(_S, _P, _N).astype(jnp.bfloat16)
    tprev = jnp.swapaxes(target.reshape(_B, _K, _S), 0, 1).reshape(_P, _S).astype(jnp.bfloat16)
    tcur_steps = jnp.concatenate([target[:, 1:], target[:, :1]], axis=1)
    tcur = jnp.swapaxes(tcur_steps.reshape(_B, _K, _S), 0, 1).reshape(_P, _S).astype(jnp.bfloat16)
    ubase = jnp.repeat(jnp.arange(_K, dtype=jnp.int32) * _S, _B)[:, None]
    lenp = jnp.tile(lengths.astype(jnp.int32), _K)[:, None]
    out = pl.pallas_call(
        _crf_body,
        out_shape=jax.ShapeDtypeStruct((_B, 1), jnp.float32),
        scratch_shapes=[pltpu.VMEM((_P, _N, _N), jnp.bfloat16),
                        pltpu.VMEM((_P, 1), jnp.float32)],
    )(lp3, lps, log_potentials[:, 0, :], tprev, tcur, target[:, :1],
      ubase, lenp, transition, start_transition[None, :],
      end_transition[None, :])
    return out[:, 0]


# speedup vs baseline: 13.0313x; 1.1649x over previous
import functools

import jax
import jax.numpy as jnp
from jax import lax
from jax.experimental import pallas as pl
from jax.experimental.pallas import tpu as pltpu

_B, _T, _N = 16, 2048, 32
_K, _S = 128, 16
_P = _K * _B
_NBLK = 2
_P2 = _P // _NBLK
_LOGC = 4.0


def _crf_body(lpsp_ref, tpv_ref, lp0_ref, tgt0_ref, ubase_ref, lenp_ref,
              tr_ref, st_ref, en_ref, out_ref, m_ref):
    f32 = jnp.float32
    bf16 = jnp.bfloat16
    E = jnp.exp(tr_ref[...])
    Tbf = tr_ref[...].astype(bf16)
    en = en_ref[...]

    Et = jnp.concatenate([E.astype(bf16)] * 4, axis=0)
    Et = jnp.concatenate([Et] * 4, axis=1)
    ri = lax.broadcasted_iota(jnp.int32, (128, 128), 0)
    li = lax.broadcasted_iota(jnp.int32, (128, 128), 1)
    Ebig = jnp.where((ri // _N) == (li // _N), Et, jnp.zeros((), bf16))

    ihi3 = lax.broadcasted_iota(jnp.int32, (_P2, 8, 128), 1)
    ln3 = lax.broadcasted_iota(jnp.int32, (_P2, 8, 128), 2)
    eyep = ((ihi3 * 4 + ln3 // _N) == (ln3 % _N)).astype(bf16)

    iota2bf = lax.broadcasted_iota(jnp.int32, (_P2, _N), 1).astype(bf16)

    lp0 = lp0_ref[...]
    iota0 = lax.broadcasted_iota(jnp.int32, (_B, _N), 1)
    oh0 = (tgt0_ref[...] == iota0).astype(f32)
    start_sel = jnp.sum(st_ref[...] * oh0, axis=1, keepdims=True)
    em0 = jnp.sum(lp0 * oh0, axis=1, keepdims=True)

    def block(blk, acc):
        em_sum, tr_sum, end_sel = acc
        r0 = blk * _P2
        ubase = ubase_ref[pl.ds(r0, _P2), :]
        lenp = lenp_ref[pl.ds(r0, _P2), :]

        def step_parts(s, tc_s, carry):
            Mbf, em_acc, tr_acc, end_acc = carry
            lp_s = lpsp_ref[s, pl.ds(r0, _P2), :].astype(f32)
            tp_s = tpv_ref[s, pl.ds(r0, _P2), :]
            ohp = (tp_s == iota2bf).astype(bf16)
            ohc = (tc_s == iota2bf).astype(bf16)
            vs = (ubase + s + 1) < lenp
            valf = vs.astype(f32)
            lastf = ((ubase + s) == (lenp - 1)).astype(f32)

            em_acc = em_acc + lp_s * ohc.astype(f32) * valf
            rows = lax.dot(ohp, Tbf, preferred_element_type=f32)
            tr_acc = tr_acc + rows * ohc.astype(f32) * valf
            end_acc = end_acc + ohp.astype(f32) * lastf

            d = jnp.exp(lp_s - _LOGC)
            dp = jnp.concatenate([d] * 4, axis=1)
            R = lax.dot(Mbf.reshape(_P2 * 8, 128), Ebig,
                        preferred_element_type=f32).reshape(_P2, 8, 128)
            R = R * dp[:, None, :]
            Mbf = jnp.where(vs[:, :, None], R.astype(bf16), Mbf)
            return Mbf, em_acc, tr_acc, end_acc

        def step(s, carry):
            tc_s = tpv_ref[s + 1, pl.ds(r0, _P2), :]
            return step_parts(s, tc_s, carry)

        z2 = jnp.zeros((_P2, _N), f32)
        carry = lax.fori_loop(0, _S - 1, step, (eyep, z2, z2, z2))
        tc_last = tpv_ref[0, pl.ds(r0 + _B, _P2), :]
        Mbf, em_acc, tr_acc, end_acc = step_parts(_S - 1, tc_last, carry)

        m_ref[pl.ds(r0 * 8, _P2 * 8)] = Mbf.reshape(_P2 * 8, 128)

        def _per_b(x2):
            xp = jnp.sum(x2, axis=1).reshape(_P2 // _B, _B)
            return jnp.sum(xp, axis=0)[:, None]

        em_sum = em_sum + _per_b(em_acc)
        tr_sum = tr_sum + _per_b(tr_acc)
        end_sel = end_sel + _per_b(end_acc * en)
        return em_sum, tr_sum, end_sel

    zero = jnp.zeros((_B, 1), f32)
    em_sum, tr_sum, end_sel = lax.fori_loop(
        0, _NBLK, block, (zero, zero, zero))

    lens = lenp_ref[pl.ds(0, _B), :]
    alpha0 = st_ref[...] + lp0

    r8 = lax.broadcasted_iota(jnp.int32, (128, 256), 0)
    l8 = lax.broadcasted_iota(jnp.int32, (128, 256), 1)
    bmask = ((r8 % 8) == (l8 // _N)).astype(bf16)
    rq = lax.broadcasted_iota(jnp.int32, (256, 128), 0)
    lq = lax.broadcasted_iota(jnp.int32, (256, 128), 1)
    Q = ((rq % _N) == ((rq // _N) * 4 + lq // _N)).astype(bf16)
    rs = lax.broadcasted_iota(jnp.int32, (128, _N), 0)
    ls = lax.broadcasted_iota(jnp.int32, (128, _N), 1)
    SS = ((rs % _N) == ls).astype(f32)

    def fold(c, alpha):
        Mcp = m_ref[pl.ds(c * 128, 128)].astype(f32)
        nval = jnp.clip(lens - 1 - c * _S, 0, _S).astype(f32)
        m = jnp.max(alpha, axis=1, keepdims=True)
        e = (jnp.exp(alpha - m)).astype(bf16)
        et = jnp.concatenate([e] * 8, axis=1)
        ebd = jnp.broadcast_to(et[:, None, :],
                               (_B, 8, 256)).reshape(128, 256) * bmask
        eexp = lax.dot(ebd, Q, preferred_element_type=f32)
        X = jnp.sum((eexp * Mcp).reshape(_B, 8, 128), axis=1)
        prod = lax.dot(X, SS, preferred_element_type=f32)
        return jnp.log(prod) + m + nval * _LOGC

    alpha = lax.fori_loop(0, _K, fold, alpha0)

    z = alpha + en
    m2 = jnp.max(z, axis=1, keepdims=True)
    logZ = jnp.log(jnp.sum(jnp.exp(z - m2), axis=1, keepdims=True)) + m2

    out_ref[...] = start_sel + em0 + em_sum + tr_sum + end_sel - logZ


@functools.partial(jax.jit, static_argnames=())
def kernel(log_potentials, target, lengths, transition, start_transition,
           end_transition):
    lp_steps = jnp.concatenate(
        [log_potentials[:, 1:, :], log_potentials[:, :1, :]], axis=1)
    lp4 = lp_steps.reshape(_B, _K, _S, _N).astype(jnp.bfloat16)
    lpsp = jnp.transpose(lp4, (2, 1, 0, 3)).reshape(_S, _P, _N)
    tg3 = jnp.transpose(target.reshape(_B, _K, _S), (2, 1, 0))
    tpv = jnp.broadcast_to(
        tg3.reshape(_S, _P, 1).astype(jnp.bfloat16), (_S, _P, _N))
    tpv = jnp.pad(tpv, ((0, 0), (0, _B), (0, 0)))
    ubase = ((jnp.arange(_P, dtype=jnp.int32) // _B) * _S)[:, None]
    lenp = jnp.tile(lengths.astype(jnp.int32), _K)[:, None]
    out = pl.pallas_call(
        _crf_body,
        out_shape=jax.ShapeDtypeStruct((_B, 1), jnp.float32),
        scratch_shapes=[pltpu.VMEM((_P * 8, 128), jnp.bfloat16)],
    )(lpsp, tpv, log_potentials[:, 0, :], target[:, :1], ubase, lenp,
      transition, start_transition[None, :], end_transition[None, :])
    return out[:, 0]
